# baseline (speedup 1.0000x reference)
import jax
import jax.numpy as jnp
from jax import lax
from jax.experimental import pallas as pl
from jax.experimental.pallas import tpu as pltpu

GAMMA = 0.01618
K_STATIC = 128
NEG_SLOPE = 0.01
OUT_DIM = 8192
ROWS = 8
COLS = OUT_DIM // ROWS
SIGN32 = -2147483648


def _mm_kernel(x_ref, w_ref, b_ref, o_ref):
    h = lax.dot_general(
        x_ref[...], w_ref[...],
        dimension_numbers=(((1,), (1,)), ((), ())),
        preferred_element_type=jnp.float32,
    )
    h = h + b_ref[...]
    o_ref[...] = jnp.where(h >= 0, h, NEG_SLOPE * h)


def _matmul(x, W, b2):
    tj = 2048
    grid = OUT_DIM // tj
    return pl.pallas_call(
        _mm_kernel,
        grid=(grid,),
        in_specs=[
            pl.BlockSpec((64, 2048), lambda j: (0, 0)),
            pl.BlockSpec((tj, 2048), lambda j: (j, 0)),
            pl.BlockSpec((1, tj), lambda j: (0, j)),
        ],
        out_specs=pl.BlockSpec((64, tj), lambda j: (0, j)),
        out_shape=jax.ShapeDtypeStruct((64, OUT_DIM), jnp.float32),
    )(x, W, b2)


def _scan_kernel(h_ref, o_ref):
    sign32 = jnp.full((1, 1), SIGN32, jnp.int32)
    idx = (lax.broadcasted_iota(jnp.int32, (ROWS, COLS), 0) * COLS
           + lax.broadcasted_iota(jnp.int32, (ROWS, COLS), 1))

    def count_ge(arr, t11):
        pred = (arr >= t11).astype(jnp.int32)
        return jnp.sum(pred, axis=(0, 1), keepdims=True)

    WIN = 1 << 19

    def step(t, carry):
        phi, pu = carry
        s = h_ref[t] * phi

        ibits = lax.bitcast_convert_type(s, jnp.int32)
        skey = ibits ^ (lax.shift_right_arithmetic(ibits, 31)
                        & jnp.int32(0x7FFFFFFF))

        lo = pu - jnp.int32(WIN)
        hi = pu + jnp.int32(WIN)
        lo_s = lo ^ sign32
        hi_s = hi ^ sign32
        c_lo = count_ge(skey, lo_s)
        c_hi = count_ge(skey, hi_s)
        pu_s = pu ^ sign32
        no_wrap_lo = pu_s >= jnp.int32(WIN - 2**31)
        no_wrap_hi = pu_s < jnp.int32(2**31 - WIN)
        good = (no_wrap_lo & no_wrap_hi
                & (c_lo >= K_STATIC) & (c_hi < K_STATIC)).astype(jnp.int32)

        def fast_select():
            inr_lo = skey >= lo_s
            inr_hi = skey >= hi_s
            diff = skey - lo_s
            d_el = jnp.where(inr_hi, jnp.int32(2 * WIN - 1),
                             jnp.where(inr_lo, diff, 0))
            du = jnp.zeros((1, 1), jnp.int32)
            for shift in (16, 12, 8, 4, 0):
                top = 16
                digit = jnp.zeros((1, 1), jnp.int32)
                for j in range(1, top):
                    cand = du | jnp.int32(j << shift)
                    cnt = count_ge(d_el, cand)
                    digit += (cnt >= K_STATIC).astype(jnp.int32)
                du = du | lax.shift_left(digit, shift)
            return lo + du

        def slow_select():
            tu = jnp.zeros((1, 1), jnp.int32)
            for shift in range(28, -1, -4):
                digit = jnp.zeros((1, 1), jnp.int32)
                for j in range(1, 16):
                    cand = tu | lax.shift_left(
                        jnp.full((1, 1), j, jnp.int32), shift)
                    cnt = count_ge(skey, cand ^ sign32)
                    digit += (cnt >= K_STATIC).astype(jnp.int32)
                tu = tu | lax.shift_left(digit, shift)
            return tu

        tu = lax.cond(good[0, 0] == 1, fast_select, slow_select)
        T = tu ^ sign32

        gt = skey > T
        eq = skey == T
        n_gt = jnp.sum(gt.astype(jnp.int32), axis=(0, 1), keepdims=True)
        need = K_STATIC - n_gt
        n_eq = jnp.sum(eq.astype(jnp.int32), axis=(0, 1), keepdims=True)

        def no_tie():
            return (skey >= T).astype(jnp.float32)

        def tie_break():
            eqi = eq.astype(jnp.int32)
            m = jnp.zeros((1, 1), jnp.int32)
            for shift in (9, 5, 1, 0):
                top = 16 if shift else 2
                digit = jnp.zeros((1, 1), jnp.int32)
                for j in range(1, top):
                    cand = m + jnp.int32(j << shift)
                    pred = eqi & (idx < cand).astype(jnp.int32)
                    cnt = jnp.sum(pred, axis=(0, 1), keepdims=True)
                    digit += (cnt < need).astype(jnp.int32)
                m = m + lax.shift_left(digit, shift)
            return (gt | (eq & (idx <= m))).astype(jnp.float32)

        maskf = lax.cond(n_eq[0, 0] == need[0, 0], no_tie, tie_break)
        o_ref[t] = maskf

        phi = jnp.where(phi < 1.0, phi + GAMMA, phi)
        phi = jnp.where(phi >= 1.0, 1.0, phi)
        return phi * (1.0 - maskf), tu

    lax.fori_loop(0, 64, step,
                  (jnp.ones((ROWS, COLS), jnp.float32),
                   jnp.zeros((1, 1), jnp.int32)))


def kernel(x, W, b, k):
    h = _matmul(x, W, b.reshape(1, OUT_DIM))
    hr = h.reshape(64, ROWS, COLS)
    mask = pl.pallas_call(
        _scan_kernel,
        in_specs=[pl.BlockSpec((64, ROWS, COLS), lambda: (0, 0, 0))],
        out_specs=pl.BlockSpec((64, ROWS, COLS), lambda: (0, 0, 0)),
        out_shape=jax.ShapeDtypeStruct((64, ROWS, COLS), jnp.float32),
    )(hr)
    k_unit = jnp.asarray(k // k, jnp.float32)
    return mask.reshape(64, OUT_DIM) * k_unit


# speedup vs baseline: 1.2025x; 1.2025x over previous
import jax
import jax.numpy as jnp
from jax import lax
from jax.experimental import pallas as pl

GAMMA = 0.01618
K_STATIC = 128
NEG_SLOPE = 0.01
OUT_DIM = 8192
ROWS = 8
COLS = OUT_DIM // ROWS
SIGN32 = -2147483648


def _mm_kernel(x_ref, w_ref, b_ref, o_ref):
    h = lax.dot_general(
        x_ref[...], w_ref[...],
        dimension_numbers=(((1,), (1,)), ((), ())),
        preferred_element_type=jnp.float32,
    )
    h = h + b_ref[...]
    o_ref[...] = jnp.where(h >= 0, h, NEG_SLOPE * h)


def _matmul(x, W, b2):
    tj = 1024
    grid = OUT_DIM // tj
    return pl.pallas_call(
        _mm_kernel,
        grid=(grid,),
        in_specs=[
            pl.BlockSpec((64, 2048), lambda j: (0, 0)),
            pl.BlockSpec((tj, 2048), lambda j: (j, 0)),
            pl.BlockSpec((1, tj), lambda j: (0, j)),
        ],
        out_specs=pl.BlockSpec((64, tj), lambda j: (0, j)),
        out_shape=jax.ShapeDtypeStruct((64, OUT_DIM), jnp.float32),
    )(x, W, b2)


def _scan_kernel(h_ref, o_ref):
    sign32 = jnp.full((1, 1), SIGN32, jnp.int32)
    idx = (lax.broadcasted_iota(jnp.int32, (ROWS, COLS), 0) * COLS
           + lax.broadcasted_iota(jnp.int32, (ROWS, COLS), 1))

    def count_ge(arr, t11):
        pred = (arr >= t11).astype(jnp.float32)
        return jnp.sum(pred, axis=(0, 1), keepdims=True)

    WIN = 1 << 19

    def step(t, carry):
        phi, pu = carry
        s = h_ref[t] * phi

        ibits = lax.bitcast_convert_type(s, jnp.int32)
        skey = ibits ^ (lax.shift_right_arithmetic(ibits, 31)
                        & jnp.int32(0x7FFFFFFF))

        lo = pu - jnp.int32(WIN)
        hi = pu + jnp.int32(WIN)
        lo_s = lo ^ sign32
        hi_s = hi ^ sign32
        c_lo = count_ge(skey, lo_s)
        c_hi = count_ge(skey, hi_s)
        pu_s = pu ^ sign32
        no_wrap_lo = pu_s >= jnp.int32(WIN - 2**31)
        no_wrap_hi = pu_s < jnp.int32(2**31 - WIN)
        good = (no_wrap_lo & no_wrap_hi
                & (c_lo >= K_STATIC) & (c_hi < K_STATIC)).astype(jnp.int32)

        inr_lo = skey >= lo_s
        inr_hi = skey >= hi_s
        diff = skey - lo_s
        d_el = jnp.where(inr_hi, jnp.int32(2 * WIN - 1),
                         jnp.where(inr_lo, diff, 0))
        digit0 = jnp.zeros((1, 1), jnp.int32)
        cg0 = c_lo
        cnts0 = []
        for j in range(1, 16):
            cnt = count_ge(d_el, jnp.int32(j << 16))
            cnts0.append(cnt)
            digit0 += (cnt >= K_STATIC).astype(jnp.int32)
        for j in range(1, 16):
            cg0 = jnp.where(digit0 == j, cnts0[j - 1], cg0)
        du0 = lax.shift_left(digit0, 16)

        def fast_select():
            du = du0
            cg = cg0
            for shift in (12, 8, 4, 0):
                digit = jnp.zeros((1, 1), jnp.int32)
                cnts = []
                for j in range(1, 16):
                    cand = du | jnp.int32(j << shift)
                    cnt = count_ge(d_el, cand)
                    cnts.append(cnt)
                    digit += (cnt >= K_STATIC).astype(jnp.int32)
                for j in range(1, 16):
                    cg = jnp.where(digit == j, cnts[j - 1], cg)
                du = du | lax.shift_left(digit, shift)
            return lo + du, cg

        def slow_select():
            tu = jnp.zeros((1, 1), jnp.int32)
            cg = jnp.full((1, 1), ROWS * COLS, jnp.float32)
            for shift in range(28, -1, -4):
                digit = jnp.zeros((1, 1), jnp.int32)
                cnts = []
                for j in range(1, 16):
                    cand = tu | lax.shift_left(
                        jnp.full((1, 1), j, jnp.int32), shift)
                    cnt = count_ge(skey, cand ^ sign32)
                    cnts.append(cnt)
                    digit += (cnt >= K_STATIC).astype(jnp.int32)
                for j in range(1, 16):
                    cg = jnp.where(digit == j, cnts[j - 1], cg)
                tu = tu | lax.shift_left(digit, shift)
            return tu, cg

        tu, c_ge = lax.cond(good[0, 0] == 1, fast_select, slow_select)
        T = tu ^ sign32


        def no_tie():
            return (skey >= T).astype(jnp.float32)

        def tie_break():
            gt = skey > T
            eq = skey == T
            n_gt = jnp.sum(gt.astype(jnp.int32), axis=(0, 1),
                           keepdims=True)
            need = K_STATIC - n_gt
            eqi = eq.astype(jnp.int32)
            m = jnp.zeros((1, 1), jnp.int32)
            for shift in (9, 5, 1, 0):
                top = 16 if shift else 2
                digit = jnp.zeros((1, 1), jnp.int32)
                for j in range(1, top):
                    cand = m + jnp.int32(j << shift)
                    pred = eqi & (idx < cand).astype(jnp.int32)
                    cnt = jnp.sum(pred, axis=(0, 1), keepdims=True)
                    digit += (cnt < need).astype(jnp.int32)
                m = m + lax.shift_left(digit, shift)
            return (gt | (eq & (idx <= m))).astype(jnp.float32)

        maskf = lax.cond(c_ge[0, 0] == K_STATIC, no_tie, tie_break)
        o_ref[t] = maskf

        phi = jnp.where(phi < 1.0, phi + GAMMA, phi)
        phi = jnp.where(phi >= 1.0, 1.0, phi)
        return phi * (1.0 - maskf), tu

    lax.fori_loop(0, 64, step,
                  (jnp.ones((ROWS, COLS), jnp.float32),
                   jnp.zeros((1, 1), jnp.int32)))


def kernel(x, W, b, k):
    h = _matmul(x, W, b.reshape(1, OUT_DIM))
    hr = h.reshape(64, ROWS, COLS)
    mask = pl.pallas_call(
        _scan_kernel,
        in_specs=[pl.BlockSpec((64, ROWS, COLS), lambda: (0, 0, 0))],
        out_specs=pl.BlockSpec((64, ROWS, COLS), lambda: (0, 0, 0)),
        out_shape=jax.ShapeDtypeStruct((64, ROWS, COLS), jnp.float32),
    )(hr)
    k_unit = jnp.asarray(k // k, jnp.float32)
    return mask.reshape(64, OUT_DIM) * k_unit
